# baseline (speedup 1.0000x reference)
import functools

import jax
import jax.numpy as jnp
from jax import lax
from jax.experimental import pallas as pl
from jax.experimental.pallas import tpu as pltpu
from jax.experimental.pallas import tpu_sc as plsc


def _make_sc_add(B, S, D):
    info = plsc.get_sparse_core_info()
    NC, NS, L = info.num_cores, info.num_subcores, info.num_lanes
    NW = NC * NS
    rows_per_w = S // NW
    CS = 64
    n_chunks = rows_per_w // CS
    vecs_per_row = D // L
    UNROLL = 8

    mesh = plsc.VectorSubcoreMesh(core_axis_name="c", subcore_axis_name="s")

    @functools.partial(
        pl.kernel,
        mesh=mesh,
        out_type=jax.ShapeDtypeStruct((B, S, D), jnp.float32),
        scratch_types=[
            pltpu.VMEM((CS, D), jnp.float32),
            pltpu.VMEM((CS, D), jnp.float32),
        ],
    )
    def sc_add(x_hbm, pos_hbm, out_hbm, buf, posbuf):
        wid = lax.axis_index("s") * NC + lax.axis_index("c")
        base = wid * rows_per_w

        n_vec = CS * vecs_per_row

        def add_block(iv, _):
            for u in range(UNROLL):
                i = iv * UNROLL + u
                r = i // vecs_per_row
                col = (i % vecs_per_row) * L
                buf[r, pl.ds(col, L)] = (
                    buf[r, pl.ds(col, L)] + posbuf[r, pl.ds(col, L)]
                )
            return _

        for c in range(n_chunks):
            row0 = base + c * CS
            pltpu.sync_copy(pos_hbm.at[pl.ds(row0, CS)], posbuf)
            for b in range(B):
                pltpu.sync_copy(x_hbm.at[b, pl.ds(row0, CS)], buf)
                lax.fori_loop(0, n_vec // UNROLL, add_block, None)
                pltpu.sync_copy(buf, out_hbm.at[b, pl.ds(row0, CS)])

    return sc_add


def kernel(x, pos_table):
    B, S, D = x.shape
    return _make_sc_add(B, S, D)(x, pos_table)


# speedup vs baseline: 1.3000x; 1.3000x over previous
import functools

import jax
import jax.numpy as jnp
from jax import lax
from jax.experimental import pallas as pl
from jax.experimental.pallas import tpu as pltpu
from jax.experimental.pallas import tpu_sc as plsc


def _make_sc_add(B, S, D):
    info = plsc.get_sparse_core_info()
    NC, NS, L = info.num_cores, info.num_subcores, info.num_lanes
    NW = NC * NS
    rows_per_w = S // NW
    CS = 16
    NB = 4
    n_chunks = rows_per_w // CS
    n_items = n_chunks * B
    vecs_per_row = D // L
    n_vec = CS * vecs_per_row
    UNROLL = 8

    mesh = plsc.VectorSubcoreMesh(core_axis_name="c", subcore_axis_name="s")

    @functools.partial(
        pl.kernel,
        mesh=mesh,
        out_type=jax.ShapeDtypeStruct((B, S, D), jnp.float32),
        scratch_types=[
            pltpu.VMEM((NB, CS, D), jnp.float32),
            pltpu.VMEM((2, CS, D), jnp.float32),
        ]
        + [pltpu.SemaphoreType.DMA] * (2 * NB + 2),
    )
    def sc_add(x_hbm, pos_hbm, out_hbm, xbuf, posbuf, *sems):
        ld_sems = sems[:NB]
        st_sems = sems[NB:2 * NB]
        pos_sems = sems[2 * NB:]

        wid = lax.axis_index("s") * NC + lax.axis_index("c")
        base = wid * rows_per_w

        def row0(c):
            return base + c * CS

        def add_chunk(slot, pslot):
            def body(iv, carry):
                for u in range(UNROLL):
                    i = iv * UNROLL + u
                    r = i // vecs_per_row
                    col = (i % vecs_per_row) * L
                    xbuf[slot, r, pl.ds(col, L)] = (
                        xbuf[slot, r, pl.ds(col, L)]
                        + posbuf[pslot, r, pl.ds(col, L)]
                    )
                return carry

            lax.fori_loop(0, n_vec // UNROLL, body, None)

        pos_cp = [None, None]
        pos_cp[0] = pltpu.async_copy(
            pos_hbm.at[pl.ds(row0(0), CS)], posbuf.at[0], pos_sems[0]
        )
        pos_waited = [False] * n_chunks
        load_cp = [None] * n_items
        store_cp = [None] * n_items

        for i in range(n_items + 1):
            if i < n_items:
                c, b = i // B, i % B
                if i >= NB:
                    store_cp[i - NB].wait()
                slot = i % NB
                load_cp[i] = pltpu.async_copy(
                    x_hbm.at[b, pl.ds(row0(c), CS)], xbuf.at[slot], ld_sems[slot]
                )
            if i >= 1:
                j = i - 1
                c, b = j // B, j % B
                load_cp[j].wait()
                if not pos_waited[c]:
                    pos_cp[c % 2].wait()
                    pos_waited[c] = True
                slot = j % NB
                add_chunk(slot, c % 2)
                store_cp[j] = pltpu.async_copy(
                    xbuf.at[slot], out_hbm.at[b, pl.ds(row0(c), CS)], st_sems[slot]
                )
                if b == B - 1 and c + 1 < n_chunks:
                    nxt = (c + 1) % 2
                    pos_cp[nxt] = pltpu.async_copy(
                        pos_hbm.at[pl.ds(row0(c + 1), CS)],
                        posbuf.at[nxt],
                        pos_sems[nxt],
                    )

        for j in range(max(0, n_items - NB), n_items):
            store_cp[j].wait()

    return sc_add


def kernel(x, pos_table):
    B, S, D = x.shape
    return _make_sc_add(B, S, D)(x, pos_table)
